# baseline (speedup 1.0000x reference)
import jax
import jax.numpy as jnp
from jax import lax
from jax.experimental import pallas as pl
from jax.experimental.pallas import tpu as pltpu

_N = 50
_F = 10
_DROP = _F + 1
_D = 65536
_HC = _D // 2
_BD = 16384
_NBC = _HC // _BD


def _chunk_kernel(x_ref, g_ref, xt_ref):
    j = pl.program_id(0)
    x_blk = x_ref[...]
    xt_ref[:, pl.ds(j * _BD, _BD)] = jnp.transpose(x_blk)
    part = lax.dot_general(x_blk, x_blk, (((0,), (0,)), ((), ())),
                           preferred_element_type=jnp.float32)

    @pl.when(j == 0)
    def _():
        g_ref[...] = part

    @pl.when(j > 0)
    def _():
        g_ref[...] = g_ref[...] + part


def _select_kernel(g0_ref, g1_ref, xt0_ref, xt1_ref, out_ref, sem0, sem1):
    g = g0_ref[...] + g1_ref[...]
    rows = lax.broadcasted_iota(jnp.int32, (_N, _N), 0)
    cols = lax.broadcasted_iota(jnp.int32, (_N, _N), 1)
    eye = rows == cols
    diag = jnp.where(eye, g, 0.0)
    sq_col = jnp.sum(diag, axis=1, keepdims=True)
    sq_row = jnp.sum(diag, axis=0, keepdims=True)
    d2 = jnp.maximum(sq_col + sq_row - 2.0 * g, 0.0)
    dist = jnp.sqrt(d2)

    total = jnp.sum(dist, axis=1, keepdims=True)
    rem = dist
    for _ in range(_DROP):
        m = jnp.max(rem, axis=1, keepdims=True)
        hit = rem == m
        first = jnp.min(jnp.where(hit, cols, _N), axis=1, keepdims=True)
        rem = jnp.where(cols == first, -1.0, rem)
        total = total - m
    scores = total

    mn = jnp.min(scores)
    ridx = lax.broadcasted_iota(jnp.int32, (_N, 1), 0)
    i_star = jnp.min(jnp.where(scores == mn, ridx, _N))

    c0 = pltpu.make_async_copy(
        xt0_ref.at[pl.ds(i_star, 1), :], out_ref.at[:, pl.ds(0, _HC)], sem0)
    c1 = pltpu.make_async_copy(
        xt1_ref.at[pl.ds(i_star, 1), :], out_ref.at[:, pl.ds(_HC, _HC)], sem1)
    c0.start()
    c1.start()
    c0.wait()
    c1.wait()


def _run_chunk(xc):
    return pl.pallas_call(
        _chunk_kernel,
        grid=(_NBC,),
        in_specs=[pl.BlockSpec((_BD, _N), lambda j: (j, 0))],
        out_specs=(pl.BlockSpec((_N, _N), lambda j: (0, 0)),
                   pl.BlockSpec((_N, _HC), lambda j: (0, 0))),
        out_shape=(jax.ShapeDtypeStruct((_N, _N), jnp.float32),
                   jax.ShapeDtypeStruct((_N, _HC), jnp.float32)),
    )(xc)


def kernel(input):
    x0 = input[0, :_HC, :]
    x1 = input[0, _HC:, :]
    g0, xt0 = _run_chunk(x0)
    g1, xt1 = _run_chunk(x1)
    out = pl.pallas_call(
        _select_kernel,
        in_specs=[
            pl.BlockSpec((_N, _N), lambda: (0, 0)),
            pl.BlockSpec((_N, _N), lambda: (0, 0)),
            pl.BlockSpec(memory_space=pl.ANY),
            pl.BlockSpec(memory_space=pl.ANY),
        ],
        out_specs=pl.BlockSpec((1, _D), lambda: (0, 0)),
        out_shape=jax.ShapeDtypeStruct((1, _D), jnp.float32),
        scratch_shapes=[pltpu.SemaphoreType.DMA, pltpu.SemaphoreType.DMA],
    )(g0, g1, xt0, xt1)
    return out.reshape(1, _D, 1)


# speedup vs baseline: 1.5624x; 1.5624x over previous
import jax
import jax.numpy as jnp
from jax import lax
from jax.experimental import pallas as pl
from jax.experimental.pallas import tpu as pltpu

_N = 50
_F = 10
_DROP = _F + 1
_D = 65536
_BD = 32768
_NB = _D // _BD


def _krum_kernel(x_ref, out_ref, xt_s, g_s):
    j = pl.program_id(0)
    x_blk = x_ref[...]
    xt_s[:, pl.ds(j * _BD, _BD)] = jnp.transpose(x_blk)
    part = lax.dot_general(x_blk, x_blk, (((0,), (0,)), ((), ())),
                           preferred_element_type=jnp.float32)

    @pl.when(j == 0)
    def _():
        g_s[...] = part

    @pl.when(j > 0)
    def _():
        g_s[...] = g_s[...] + part

    @pl.when(j == _NB - 1)
    def _():
        g = g_s[...]
        rows = lax.broadcasted_iota(jnp.int32, (_N, _N), 0)
        cols = lax.broadcasted_iota(jnp.int32, (_N, _N), 1)
        eye = rows == cols
        diag = jnp.where(eye, g, 0.0)
        sq_col = jnp.sum(diag, axis=1, keepdims=True)
        sq_row = jnp.sum(diag, axis=0, keepdims=True)
        d2 = jnp.maximum(sq_col + sq_row - 2.0 * g, 0.0)
        dist = jnp.sqrt(d2)

        total = jnp.sum(dist, axis=1, keepdims=True)
        rem = dist
        for _ in range(_DROP):
            m = jnp.max(rem, axis=1, keepdims=True)
            hit = rem == m
            first = jnp.min(jnp.where(hit, cols, _N), axis=1, keepdims=True)
            rem = jnp.where(cols == first, -1.0, rem)
            total = total - m
        scores = total

        mn = jnp.min(scores)
        ridx = lax.broadcasted_iota(jnp.int32, (_N, 1), 0)
        i_star = jnp.min(jnp.where(scores == mn, ridx, _N))
        out_ref[...] = xt_s[pl.ds(i_star, 1), :]


def kernel(input):
    x = input.reshape(_D, _N)
    out = pl.pallas_call(
        _krum_kernel,
        grid=(_NB,),
        in_specs=[pl.BlockSpec((_BD, _N), lambda j: (j, 0))],
        out_specs=pl.BlockSpec((1, _D), lambda j: (0, 0)),
        out_shape=jax.ShapeDtypeStruct((1, _D), jnp.float32),
        scratch_shapes=[
            pltpu.VMEM((_N, _D), jnp.float32),
            pltpu.VMEM((_N, _N), jnp.float32),
        ],
    )(x)
    return out.reshape(1, _D, 1)
